# baseline (speedup 1.0000x reference)
import functools

import jax
import jax.numpy as jnp
from jax import lax
from jax.experimental import pallas as pl
from jax.experimental.pallas import tpu as pltpu
from jax.experimental.pallas import tpu_sc as plsc

F = 26
V = 100000
D = 16
B = 4096
FD = F * D

_NC, _NS = 2, 16
_NF1 = 12
_NF2 = F - _NF1


def _sc_group(emb_t, lin2d, x_t, f0, nf):
    mesh = plsc.VectorSubcoreMesh(core_axis_name="c", subcore_axis_name="s")

    @functools.partial(
        pl.kernel,
        mesh=mesh,
        out_type=[
            jax.ShapeDtypeStruct((nf * D, B), jnp.float32),
            jax.ShapeDtypeStruct((nf, B), jnp.float32),
        ],
        scratch_types=[
            pltpu.VMEM((V,), jnp.float32),
            pltpu.VMEM((B,), jnp.int32),
            pltpu.VMEM((B,), jnp.float32),
        ],
        compiler_params=pltpu.CompilerParams(
            use_tc_tiling_on_sc=True, needs_layout_passes=False),
    )
    def k(emb_hbm, lin_hbm, x_hbm, xt_out, lt_out, plane_v, idx_v, sel_v):
        c = lax.axis_index("c")
        s = lax.axis_index("s")
        wid = s * _NC + c

        def select():
            def body(i, _):
                v16 = idx_v[pl.ds(i * 16, 16)]
                sel_v[pl.ds(i * 16, 16)] = plsc.load_gather(plane_v, [v16])
                return 0

            lax.fori_loop(0, B // 16, body, 0)

        for kf in range(nf // _NC):
            jloc = 2 * kf + c
            pltpu.sync_copy(x_hbm.at[f0 + jloc], idx_v)
            pltpu.sync_copy(emb_hbm.at[f0 + jloc, s], plane_v)
            select()
            pltpu.sync_copy(sel_v, xt_out.at[jloc * D + s])

        @pl.when(wid < nf)
        def _():
            pltpu.sync_copy(x_hbm.at[f0 + wid], idx_v)
            pltpu.sync_copy(lin_hbm.at[f0 + wid], plane_v)
            select()
            pltpu.sync_copy(sel_v, lt_out.at[wid])

    return k(emb_t, lin2d, x_t)


_BT = 1024


def _field_sum_mask(nrows):
    didx = lax.broadcasted_iota(jnp.int32, (D, nrows), 0)
    ridx = lax.broadcasted_iota(jnp.int32, (D, nrows), 1)
    return (ridx % D == didx).astype(jnp.float32)


def _tc_partial_body(xt_ref, lt_ref, w1_ref, acc_ref):
    xt = xt_ref[...]
    dnT = (((0,), (0,)), ((), ()))
    h_pre = lax.dot_general(w1_ref[...], xt, dnT)
    dn = (((1,), (0,)), ((), ()))
    s1 = lax.dot_general(_field_sum_mask(_NF1 * D), xt, dn)
    q = jnp.sum(xt * xt, axis=0, keepdims=True)
    linp = jnp.sum(lt_ref[...], axis=0, keepdims=True)
    acc_ref[...] = jnp.concatenate([h_pre, s1, q, linp], axis=0)


def _tc_partial(xt_g, lt_g, W1_g):
    return pl.pallas_call(
        _tc_partial_body,
        grid=(B // _BT,),
        in_specs=[
            pl.BlockSpec((_NF1 * D, _BT), lambda i: (0, i)),
            pl.BlockSpec((_NF1, _BT), lambda i: (0, i)),
            pl.BlockSpec((_NF1 * D, 64), lambda i: (0, 0)),
        ],
        out_specs=pl.BlockSpec((82, _BT), lambda i: (0, i)),
        out_shape=jax.ShapeDtypeStruct((82, B), jnp.float32),
    )(xt_g, lt_g, W1_g)


def _tc_final_body(xt_ref, lt_ref, acc_ref, w1_ref, b1_ref, w2_ref, b2_ref,
                   w3_ref, b3_ref, out_ref):
    xt = xt_ref[...]
    acc = acc_ref[...]
    dnT = (((0,), (0,)), ((), ()))
    h_pre = acc[0:64, :] + lax.dot_general(w1_ref[...], xt, dnT)
    h = jnp.maximum(h_pre + b1_ref[...], 0.0)
    dn = (((1,), (0,)), ((), ()))
    s1 = acc[64:80, :] + lax.dot_general(_field_sum_mask(_NF2 * D), xt, dn)
    q = acc[80, :] + jnp.sum(xt * xt, axis=0)
    lin = acc[81, :] + jnp.sum(lt_ref[...], axis=0)
    fm = 0.5 * (jnp.sum(s1 * s1, axis=0) - q)

    h = jnp.maximum(lax.dot_general(w2_ref[...], h, dnT) + b2_ref[...], 0.0)
    deep = lax.dot_general(w3_ref[...], h, dnT)[0, :] + b3_ref[0, 0]
    out_ref[...] = lin + fm + deep


def _tc_final(xt_g, lt_g, acc, W1_g, b1, W2, b2, W3, b3):
    return pl.pallas_call(
        _tc_final_body,
        grid=(B // _BT,),
        in_specs=[
            pl.BlockSpec((_NF2 * D, _BT), lambda i: (0, i)),
            pl.BlockSpec((_NF2, _BT), lambda i: (0, i)),
            pl.BlockSpec((82, _BT), lambda i: (0, i)),
            pl.BlockSpec((_NF2 * D, 64), lambda i: (0, 0)),
            pl.BlockSpec((64, 1), lambda i: (0, 0)),
            pl.BlockSpec((64, 32), lambda i: (0, 0)),
            pl.BlockSpec((32, 1), lambda i: (0, 0)),
            pl.BlockSpec((32, 1), lambda i: (0, 0)),
            pl.BlockSpec((1, 1), lambda i: (0, 0)),
        ],
        out_specs=pl.BlockSpec((_BT,), lambda i: (i,)),
        out_shape=jax.ShapeDtypeStruct((B,), jnp.float32),
    )(xt_g, lt_g, acc, W1_g, b1, W2, b2, W3, b3)


def kernel(x_cat, lin_tables, emb_tables, W1, b1, W2, b2, W3, b3):
    emb_t = jnp.transpose(emb_tables, (0, 2, 1))
    lin2d = jnp.transpose(lin_tables, (0, 2, 1)).reshape(F, V)
    x_t = jnp.transpose(x_cat.astype(jnp.int32), (1, 0))

    xt1, lt1 = _sc_group(emb_t, lin2d, x_t, 0, _NF1)
    xt2, lt2 = _sc_group(emb_t, lin2d, x_t, _NF1, _NF2)

    acc = _tc_partial(xt1, lt1, W1[: _NF1 * D])
    return _tc_final(xt2, lt2, acc, W1[_NF1 * D :], b1.reshape(64, 1),
                     W2, b2.reshape(32, 1), W3, b3.reshape(1, 1))


# speedup vs baseline: 1.1033x; 1.1033x over previous
import functools

import jax
import jax.numpy as jnp
from jax import lax
from jax.experimental import pallas as pl
from jax.experimental.pallas import tpu as pltpu
from jax.experimental.pallas import tpu_sc as plsc

F = 26
V = 100000
D = 16
B = 4096
FD = F * D

_NC, _NS = 2, 16
_H0 = 51200
_H1 = V - _H0


def _sc_gather_all(emb_t, lin2d, x_t):
    mesh = plsc.VectorSubcoreMesh(core_axis_name="c", subcore_axis_name="s")

    @functools.partial(
        pl.kernel,
        mesh=mesh,
        out_type=[
            jax.ShapeDtypeStruct((FD, B), jnp.float32),
            jax.ShapeDtypeStruct((F, B), jnp.float32),
        ],
        scratch_types=[
            pltpu.VMEM((V,), jnp.float32),
            pltpu.VMEM((B,), jnp.int32),
            pltpu.VMEM((B,), jnp.float32),
        ],
        compiler_params=pltpu.CompilerParams(
            use_tc_tiling_on_sc=True, needs_layout_passes=False),
    )
    def k(emb_hbm, lin_hbm, x_hbm, xt_out, lt_out, plane_v, idx_v, sel_v):
        c = lax.axis_index("c")
        s = lax.axis_index("s")
        wid = s * _NC + c

        def select():
            def body(i, _):
                v16 = idx_v[pl.ds(i * 16, 16)]
                sel_v[pl.ds(i * 16, 16)] = plsc.load_gather(plane_v, [v16])
                return 0

            lax.fori_loop(0, B // 16, body, 0)

        @pl.when(wid < F)
        def _():
            pltpu.sync_copy(x_hbm.at[wid], idx_v)
            pltpu.sync_copy(lin_hbm.at[wid], plane_v)
            select()
            pltpu.sync_copy(sel_v, lt_out.at[wid])

        for kf in range(F // _NC):
            f = 2 * kf + c
            pltpu.sync_copy(x_hbm.at[f], idx_v)
            pltpu.sync_copy(emb_hbm.at[f, s], plane_v)
            select()
            pltpu.sync_copy(sel_v, xt_out.at[f * D + s])

    return k(emb_t, lin2d, x_t)


_BT = 1024


def _tc_body(xt_ref, lt_ref, w1_ref, b1_ref, w2_ref, b2_ref, w3_ref, b3_ref,
             out_ref):
    xt = xt_ref[...]
    lt = lt_ref[...]
    linear_logit = jnp.sum(lt, axis=0)

    didx = lax.broadcasted_iota(jnp.int32, (D, FD), 0)
    ridx = lax.broadcasted_iota(jnp.int32, (D, FD), 1)
    R = (ridx % D == didx).astype(jnp.float32)
    dn = (((1,), (0,)), ((), ()))
    s1 = lax.dot_general(R, xt, dn)
    q = jnp.sum(xt * xt, axis=0)
    fm_logit = 0.5 * (jnp.sum(s1 * s1, axis=0) - q)

    dnT = (((0,), (0,)), ((), ()))
    h = jnp.maximum(lax.dot_general(w1_ref[...], xt, dnT) + b1_ref[...], 0.0)
    h = jnp.maximum(lax.dot_general(w2_ref[...], h, dnT) + b2_ref[...], 0.0)
    deep = lax.dot_general(w3_ref[...], h, dnT)[0, :] + b3_ref[0, 0]

    out_ref[...] = linear_logit + fm_logit + deep


def _tc_head(xt, lt, W1, b1, W2, b2, W3, b3):
    return pl.pallas_call(
        _tc_body,
        grid=(B // _BT,),
        in_specs=[
            pl.BlockSpec((FD, _BT), lambda i: (0, i)),
            pl.BlockSpec((F, _BT), lambda i: (0, i)),
            pl.BlockSpec((FD, 64), lambda i: (0, 0)),
            pl.BlockSpec((64, 1), lambda i: (0, 0)),
            pl.BlockSpec((64, 32), lambda i: (0, 0)),
            pl.BlockSpec((32, 1), lambda i: (0, 0)),
            pl.BlockSpec((32, 1), lambda i: (0, 0)),
            pl.BlockSpec((1, 1), lambda i: (0, 0)),
        ],
        out_specs=pl.BlockSpec((_BT,), lambda i: (i,)),
        out_shape=jax.ShapeDtypeStruct((B,), jnp.float32),
    )(xt, lt, W1, b1, W2, b2, W3, b3)


def kernel(x_cat, lin_tables, emb_tables, W1, b1, W2, b2, W3, b3):
    emb_t = jnp.transpose(emb_tables, (0, 2, 1))
    lin2d = jnp.transpose(lin_tables, (0, 2, 1)).reshape(F, V)
    x_t = jnp.transpose(x_cat.astype(jnp.int32), (1, 0))

    xt, lt = _sc_gather_all(emb_t, lin2d, x_t)

    return _tc_head(xt, lt, W1, b1.reshape(64, 1), W2, b2.reshape(32, 1),
                    W3, b3.reshape(1, 1))


# speedup vs baseline: 1.1059x; 1.0023x over previous
import functools

import jax
import jax.numpy as jnp
from jax import lax
from jax.experimental import pallas as pl
from jax.experimental.pallas import tpu as pltpu
from jax.experimental.pallas import tpu_sc as plsc

F = 26
V = 100000
D = 16
B = 4096
FD = F * D

_NC, _NS = 2, 16
_H0 = 51200
_H1 = V - _H0


def _sc_gather_all(emb_t, lin2d, x_t):
    mesh = plsc.VectorSubcoreMesh(core_axis_name="c", subcore_axis_name="s")

    @functools.partial(
        pl.kernel,
        mesh=mesh,
        out_type=[
            jax.ShapeDtypeStruct((FD, B), jnp.float32),
            jax.ShapeDtypeStruct((F, B), jnp.float32),
        ],
        scratch_types=[
            pltpu.VMEM((V,), jnp.float32),
            pltpu.VMEM((B,), jnp.int32),
            pltpu.VMEM((B,), jnp.float32),
        ],
        compiler_params=pltpu.CompilerParams(
            use_tc_tiling_on_sc=True, needs_layout_passes=False),
    )
    def k(emb_hbm, lin_hbm, x_hbm, xt_out, lt_out, plane_v, idx_v, sel_v):
        c = lax.axis_index("c")
        s = lax.axis_index("s")
        wid = s * _NC + c

        def select():
            def body(i, _):
                v16 = idx_v[pl.ds(i * 16, 16)]
                sel_v[pl.ds(i * 16, 16)] = plsc.load_gather(plane_v, [v16])
                return 0

            lax.fori_loop(0, B // 16, body, 0)

        @pl.when(wid < F)
        def _():
            pltpu.sync_copy(x_hbm.at[wid], idx_v)
            pltpu.sync_copy(lin_hbm.at[wid], plane_v)
            select()
            pltpu.sync_copy(sel_v, lt_out.at[wid])

        for kf in range(F // _NC):
            f = 2 * kf + c
            pltpu.sync_copy(x_hbm.at[f], idx_v)
            pltpu.sync_copy(emb_hbm.at[f, s], plane_v)
            select()
            pltpu.sync_copy(sel_v, xt_out.at[f * D + s])

    return k(emb_t, lin2d, x_t)


_BT = 1024


def _tc_body(xt_ref, lt_ref, w1_ref, b1_ref, w2_ref, b2_ref, w3_ref, b3_ref,
             out_ref):
    xt = xt_ref[...]
    lt = lt_ref[...]
    linear_logit = jnp.sum(lt, axis=0)

    didx = lax.broadcasted_iota(jnp.int32, (D, FD), 0)
    ridx = lax.broadcasted_iota(jnp.int32, (D, FD), 1)
    R = (ridx % D == didx).astype(jnp.float32)
    dn = (((1,), (0,)), ((), ()))
    s1 = lax.dot_general(R, xt, dn)
    q = jnp.sum(xt * xt, axis=0)
    fm_logit = 0.5 * (jnp.sum(s1 * s1, axis=0) - q)

    dnT = (((0,), (0,)), ((), ()))
    pt = jnp.float32
    h = jnp.maximum(
        lax.dot_general(w1_ref[...].astype(jnp.bfloat16),
                        xt.astype(jnp.bfloat16), dnT,
                        preferred_element_type=pt) + b1_ref[...], 0.0)
    h = jnp.maximum(
        lax.dot_general(w2_ref[...].astype(jnp.bfloat16),
                        h.astype(jnp.bfloat16), dnT,
                        preferred_element_type=pt) + b2_ref[...], 0.0)
    deep = lax.dot_general(w3_ref[...].astype(jnp.bfloat16),
                           h.astype(jnp.bfloat16), dnT,
                           preferred_element_type=pt)[0, :] + b3_ref[0, 0]

    out_ref[...] = linear_logit + fm_logit + deep


def _tc_head(xt, lt, W1, b1, W2, b2, W3, b3):
    return pl.pallas_call(
        _tc_body,
        grid=(B // _BT,),
        in_specs=[
            pl.BlockSpec((FD, _BT), lambda i: (0, i)),
            pl.BlockSpec((F, _BT), lambda i: (0, i)),
            pl.BlockSpec((FD, 64), lambda i: (0, 0)),
            pl.BlockSpec((64, 1), lambda i: (0, 0)),
            pl.BlockSpec((64, 32), lambda i: (0, 0)),
            pl.BlockSpec((32, 1), lambda i: (0, 0)),
            pl.BlockSpec((32, 1), lambda i: (0, 0)),
            pl.BlockSpec((1, 1), lambda i: (0, 0)),
        ],
        out_specs=pl.BlockSpec((_BT,), lambda i: (i,)),
        out_shape=jax.ShapeDtypeStruct((B,), jnp.float32),
    )(xt, lt, W1, b1, W2, b2, W3, b3)


def kernel(x_cat, lin_tables, emb_tables, W1, b1, W2, b2, W3, b3):
    emb_t = jnp.transpose(emb_tables, (0, 2, 1))
    lin2d = jnp.transpose(lin_tables, (0, 2, 1)).reshape(F, V)
    x_t = jnp.transpose(x_cat.astype(jnp.int32), (1, 0))

    xt, lt = _sc_gather_all(emb_t, lin2d, x_t)

    return _tc_head(xt, lt, W1, b1.reshape(64, 1), W2, b2.reshape(32, 1),
                    W3, b3.reshape(1, 1))


# speedup vs baseline: 1.1741x; 1.0617x over previous
import functools

import jax
import jax.numpy as jnp
from jax import lax
from jax.experimental import pallas as pl
from jax.experimental.pallas import tpu as pltpu
from jax.experimental.pallas import tpu_sc as plsc

F = 26
V = 100000
D = 16
B = 4096
FD = F * D

_NC, _NS = 2, 16
_H0 = 51200
_H1 = V - _H0


def _sc_gather_all(emb_t, lin2d, x_t):
    mesh = plsc.VectorSubcoreMesh(core_axis_name="c", subcore_axis_name="s")

    @functools.partial(
        pl.kernel,
        mesh=mesh,
        out_type=[
            jax.ShapeDtypeStruct((FD, B), jnp.float32),
            jax.ShapeDtypeStruct((F, B), jnp.float32),
        ],
        scratch_types=[
            pltpu.VMEM((V,), jnp.float32),
            pltpu.VMEM((B,), jnp.int32),
            pltpu.VMEM((B,), jnp.float32),
            pltpu.SemaphoreType.DMA,
        ],
        compiler_params=pltpu.CompilerParams(
            use_tc_tiling_on_sc=True, needs_layout_passes=False),
    )
    def k(emb_hbm, lin_hbm, x_hbm, xt_out, lt_out, plane_v, idx_v, sel_v,
          semp):
        c = lax.axis_index("c")
        s = lax.axis_index("s")
        wid = s * _NC + c

        def select():
            def body(i, _):
                v16 = idx_v[pl.ds(i * 16, 16)]
                sel_v[pl.ds(i * 16, 16)] = plsc.load_gather(plane_v, [v16])
                return 0

            lax.fori_loop(0, B // 16, body, 0)

        @pl.when(wid < F)
        def _():
            pltpu.sync_copy(x_hbm.at[wid], idx_v)
            pltpu.sync_copy(lin_hbm.at[wid], plane_v)
            select()
            pltpu.sync_copy(sel_v, lt_out.at[wid])

        for kf in range(F // _NC):
            f = 2 * kf + c
            cp = pltpu.async_copy(emb_hbm.at[f, s], plane_v, semp)
            pltpu.sync_copy(x_hbm.at[f], idx_v)
            cp.wait()
            select()
            pltpu.sync_copy(sel_v, xt_out.at[f * D + s])

    return k(emb_t, lin2d, x_t)


_BT = 1024


def _tc_body(xt_ref, lt_ref, w1_ref, b1_ref, w2_ref, b2_ref, w3_ref, b3_ref,
             out_ref):
    xt = xt_ref[...]
    lt = lt_ref[...]
    linear_logit = jnp.sum(lt, axis=0)

    didx = lax.broadcasted_iota(jnp.int32, (D, FD), 0)
    ridx = lax.broadcasted_iota(jnp.int32, (D, FD), 1)
    R = (ridx % D == didx).astype(jnp.float32)
    dn = (((1,), (0,)), ((), ()))
    s1 = lax.dot_general(R, xt, dn)
    q = jnp.sum(xt * xt, axis=0)
    fm_logit = 0.5 * (jnp.sum(s1 * s1, axis=0) - q)

    dnT = (((0,), (0,)), ((), ()))
    pt = jnp.float32
    h = jnp.maximum(
        lax.dot_general(w1_ref[...].astype(jnp.bfloat16),
                        xt.astype(jnp.bfloat16), dnT,
                        preferred_element_type=pt) + b1_ref[...], 0.0)
    h = jnp.maximum(
        lax.dot_general(w2_ref[...].astype(jnp.bfloat16),
                        h.astype(jnp.bfloat16), dnT,
                        preferred_element_type=pt) + b2_ref[...], 0.0)
    deep = lax.dot_general(w3_ref[...].astype(jnp.bfloat16),
                           h.astype(jnp.bfloat16), dnT,
                           preferred_element_type=pt)[0, :] + b3_ref[0, 0]

    out_ref[...] = linear_logit + fm_logit + deep


def _tc_head(xt, lt, W1, b1, W2, b2, W3, b3):
    return pl.pallas_call(
        _tc_body,
        grid=(B // _BT,),
        in_specs=[
            pl.BlockSpec((FD, _BT), lambda i: (0, i)),
            pl.BlockSpec((F, _BT), lambda i: (0, i)),
            pl.BlockSpec((FD, 64), lambda i: (0, 0)),
            pl.BlockSpec((64, 1), lambda i: (0, 0)),
            pl.BlockSpec((64, 32), lambda i: (0, 0)),
            pl.BlockSpec((32, 1), lambda i: (0, 0)),
            pl.BlockSpec((32, 1), lambda i: (0, 0)),
            pl.BlockSpec((1, 1), lambda i: (0, 0)),
        ],
        out_specs=pl.BlockSpec((_BT,), lambda i: (i,)),
        out_shape=jax.ShapeDtypeStruct((B,), jnp.float32),
    )(xt, lt, W1, b1, W2, b2, W3, b3)


def kernel(x_cat, lin_tables, emb_tables, W1, b1, W2, b2, W3, b3):
    emb_t = jnp.transpose(emb_tables, (0, 2, 1))
    lin2d = jnp.transpose(lin_tables, (0, 2, 1)).reshape(F, V)
    x_t = jnp.transpose(x_cat.astype(jnp.int32), (1, 0))

    xt, lt = _sc_gather_all(emb_t, lin2d, x_t)

    return _tc_head(xt, lt, W1, b1.reshape(64, 1), W2, b2.reshape(32, 1),
                    W3, b3.reshape(1, 1))


# speedup vs baseline: 1.2035x; 1.0251x over previous
import functools

import jax
import jax.numpy as jnp
from jax import lax
from jax.experimental import pallas as pl
from jax.experimental.pallas import tpu as pltpu
from jax.experimental.pallas import tpu_sc as plsc

F = 26
V = 100000
D = 16
B = 4096
FD = F * D

_NC, _NS = 2, 16
_H0 = 51200
_H1 = V - _H0


def _sc_gather_all(emb_t, lin2d, x_t):
    mesh = plsc.VectorSubcoreMesh(core_axis_name="c", subcore_axis_name="s")

    @functools.partial(
        pl.kernel,
        mesh=mesh,
        out_type=[
            jax.ShapeDtypeStruct((FD, B), jnp.float32),
            jax.ShapeDtypeStruct((F, B), jnp.float32),
        ],
        scratch_types=[
            pltpu.VMEM((V,), jnp.float32),
            pltpu.VMEM((B,), jnp.int32),
            pltpu.VMEM((B,), jnp.float32),
            pltpu.VMEM((B,), jnp.float32),
            pltpu.SemaphoreType.DMA,
            pltpu.SemaphoreType.DMA,
            pltpu.SemaphoreType.DMA,
        ],
        compiler_params=pltpu.CompilerParams(
            use_tc_tiling_on_sc=True, needs_layout_passes=False),
    )
    def k(emb_hbm, lin_hbm, x_hbm, xt_out, lt_out, plane_v, idx_v, sel_a,
          sel_b, semp, semw0, semw1):
        c = lax.axis_index("c")
        s = lax.axis_index("s")
        wid = s * _NC + c
        sel = (sel_a, sel_b)
        semw = (semw0, semw1)

        def select(sel_v):
            def body(i, _):
                v16 = idx_v[pl.ds(i * 16, 16)]
                sel_v[pl.ds(i * 16, 16)] = plsc.load_gather(plane_v, [v16])
                return 0

            lax.fori_loop(0, B // 16, body, 0)

        @pl.when(wid < F)
        def _():
            pltpu.sync_copy(x_hbm.at[wid], idx_v)
            pltpu.sync_copy(lin_hbm.at[wid], plane_v)
            select(sel_a)
            pltpu.sync_copy(sel_a, lt_out.at[wid])

        wcs = [None, None]
        for kf in range(F // _NC):
            f = 2 * kf + c
            cp = pltpu.async_copy(emb_hbm.at[f, s], plane_v, semp)
            pltpu.sync_copy(x_hbm.at[f], idx_v)
            j = kf & 1
            if wcs[j] is not None:
                wcs[j].wait()
            cp.wait()
            select(sel[j])
            wcs[j] = pltpu.async_copy(sel[j], xt_out.at[f * D + s], semw[j])
        for wc in wcs:
            if wc is not None:
                wc.wait()

    return k(emb_t, lin2d, x_t)


_BT = 1024


def _tc_body(xt_ref, lt_ref, w1_ref, b1_ref, w2_ref, b2_ref, w3_ref, b3_ref,
             out_ref):
    xt = xt_ref[...]
    lt = lt_ref[...]
    linear_logit = jnp.sum(lt, axis=0)

    didx = lax.broadcasted_iota(jnp.int32, (D, FD), 0)
    ridx = lax.broadcasted_iota(jnp.int32, (D, FD), 1)
    R = (ridx % D == didx).astype(jnp.float32)
    dn = (((1,), (0,)), ((), ()))
    s1 = lax.dot_general(R, xt, dn)
    q = jnp.sum(xt * xt, axis=0)
    fm_logit = 0.5 * (jnp.sum(s1 * s1, axis=0) - q)

    dnT = (((0,), (0,)), ((), ()))
    pt = jnp.float32
    h = jnp.maximum(
        lax.dot_general(w1_ref[...].astype(jnp.bfloat16),
                        xt.astype(jnp.bfloat16), dnT,
                        preferred_element_type=pt) + b1_ref[...], 0.0)
    h = jnp.maximum(
        lax.dot_general(w2_ref[...].astype(jnp.bfloat16),
                        h.astype(jnp.bfloat16), dnT,
                        preferred_element_type=pt) + b2_ref[...], 0.0)
    deep = lax.dot_general(w3_ref[...].astype(jnp.bfloat16),
                           h.astype(jnp.bfloat16), dnT,
                           preferred_element_type=pt)[0, :] + b3_ref[0, 0]

    out_ref[...] = linear_logit + fm_logit + deep


def _tc_head(xt, lt, W1, b1, W2, b2, W3, b3):
    return pl.pallas_call(
        _tc_body,
        grid=(B // _BT,),
        in_specs=[
            pl.BlockSpec((FD, _BT), lambda i: (0, i)),
            pl.BlockSpec((F, _BT), lambda i: (0, i)),
            pl.BlockSpec((FD, 64), lambda i: (0, 0)),
            pl.BlockSpec((64, 1), lambda i: (0, 0)),
            pl.BlockSpec((64, 32), lambda i: (0, 0)),
            pl.BlockSpec((32, 1), lambda i: (0, 0)),
            pl.BlockSpec((32, 1), lambda i: (0, 0)),
            pl.BlockSpec((1, 1), lambda i: (0, 0)),
        ],
        out_specs=pl.BlockSpec((_BT,), lambda i: (i,)),
        out_shape=jax.ShapeDtypeStruct((B,), jnp.float32),
    )(xt, lt, W1, b1, W2, b2, W3, b3)


def kernel(x_cat, lin_tables, emb_tables, W1, b1, W2, b2, W3, b3):
    emb_t = jnp.transpose(emb_tables, (0, 2, 1))
    lin2d = jnp.transpose(lin_tables, (0, 2, 1)).reshape(F, V)
    x_t = jnp.transpose(x_cat.astype(jnp.int32), (1, 0))

    xt, lt = _sc_gather_all(emb_t, lin2d, x_t)

    return _tc_head(xt, lt, W1, b1.reshape(64, 1), W2, b2.reshape(32, 1),
                    W3, b3.reshape(1, 1))
